# baseline (speedup 1.0000x reference)
import functools

import jax
import jax.numpy as jnp
from jax import lax
from jax.experimental import pallas as pl
from jax.experimental.pallas import tpu as pltpu
from jax.experimental.pallas import tpu_sc as plsc

VOCAB = 1000000
MAXLEN = 200
EMBED_DIM = 64
BATCH = 4096

NUM_CORES = 2
NUM_SUBCORES = 16
NUM_WORKERS = NUM_CORES * NUM_SUBCORES
NROWS = BATCH * MAXLEN
ROWS_PER_WORKER = NROWS // NUM_WORKERS
SEQS_PER_CHUNK = 4
CHUNK = SEQS_PER_CHUNK * MAXLEN
CHUNKS_PER_WORKER = ROWS_PER_WORKER // CHUNK
SUBGATHERS = tuple((o, min(128, CHUNK - o)) for o in range(0, CHUNK, 128))


@functools.partial(
    pl.kernel,
    out_type=jax.ShapeDtypeStruct((NROWS, EMBED_DIM), jnp.float32),
    mesh=plsc.VectorSubcoreMesh(core_axis_name="c", subcore_axis_name="s"),
    scratch_types=[
        pltpu.VMEM((CHUNK,), jnp.int32),
        pltpu.VMEM((CHUNK,), jnp.int32),
        pltpu.VMEM((CHUNK, EMBED_DIM), jnp.float32),
        pltpu.SemaphoreType.DMA,
    ],
    compiler_params=pltpu.CompilerParams(use_tc_tiling_on_sc=False),
)
def _emb_kernel(x_hbm, tok_hbm, pos_hbm, pidx_hbm, out_hbm,
                idx_v, pidx_v, rows_v, sem):
    wid = lax.axis_index("s") * NUM_CORES + lax.axis_index("c")
    base = wid * ROWS_PER_WORKER

    pltpu.sync_copy(pidx_hbm, pidx_v)

    def chunk_body(k, carry):
        row0 = base + k * CHUNK
        pltpu.sync_copy(x_hbm.at[pl.ds(row0, CHUNK)], idx_v)
        handles = []
        for sb, sl in SUBGATHERS:
            handles.append(
                pltpu.async_copy(
                    tok_hbm.at[idx_v.at[pl.ds(sb, sl)]],
                    rows_v.at[pl.ds(sb, sl)],
                    sem,
                )
            )
        for h in handles:
            h.wait()
        handles = []
        for sb, sl in SUBGATHERS:
            handles.append(
                pltpu.async_copy(
                    pos_hbm.at[pidx_v.at[pl.ds(sb, sl)]],
                    rows_v.at[pl.ds(sb, sl)],
                    sem,
                    add=True,
                )
            )
        for h in handles:
            h.wait()
        pltpu.sync_copy(rows_v, out_hbm.at[pl.ds(row0, CHUNK)])
        return carry

    lax.fori_loop(0, CHUNKS_PER_WORKER, chunk_body, 0)


def kernel(x, token_emb, pos_emb):
    x_flat = x.reshape(-1).astype(jnp.int32)
    pidx = jnp.tile(jnp.arange(MAXLEN, dtype=jnp.int32), SEQS_PER_CHUNK)
    out = _emb_kernel(x_flat, token_emb, pos_emb, pidx)
    return out.reshape(BATCH, MAXLEN, EMBED_DIM)


# speedup vs baseline: 1.2687x; 1.2687x over previous
import functools

import jax
import jax.numpy as jnp
from jax import lax
from jax.experimental import pallas as pl
from jax.experimental.pallas import tpu as pltpu
from jax.experimental.pallas import tpu_sc as plsc

VOCAB = 1000000
MAXLEN = 200
EMBED_DIM = 64
BATCH = 4096

NUM_CORES = 2
NUM_SUBCORES = 16
NUM_WORKERS = NUM_CORES * NUM_SUBCORES
NROWS = BATCH * MAXLEN
ROWS_PER_WORKER = NROWS // NUM_WORKERS
SEQS_PER_WORKER = ROWS_PER_WORKER // MAXLEN
RING = 4
SUBS = ((0, 128), (128, 72))
LANES = 16
DBLK = EMBED_DIM // LANES


@functools.partial(
    pl.kernel,
    out_type=jax.ShapeDtypeStruct((NROWS, EMBED_DIM), jnp.float32),
    mesh=plsc.VectorSubcoreMesh(core_axis_name="c", subcore_axis_name="s"),
    scratch_types=[
        pltpu.VMEM((ROWS_PER_WORKER,), jnp.int32),
        pltpu.VMEM((MAXLEN, EMBED_DIM), jnp.float32),
    ]
    + [pltpu.VMEM((MAXLEN, EMBED_DIM), jnp.float32) for _ in range(RING)]
    + [pltpu.SemaphoreType.DMA for _ in range(2 * RING)],
    compiler_params=pltpu.CompilerParams(use_tc_tiling_on_sc=False),
)
def _emb_kernel(x_hbm, tok_hbm, pos_hbm, out_hbm, idx_v, pos_v,
                r0, r1, r2, r3, g0, g1, g2, g3, o0, o1, o2, o3):
    ring = (r0, r1, r2, r3)
    gsem = (g0, g1, g2, g3)
    osem = (o0, o1, o2, o3)
    wid = lax.axis_index("s") * NUM_CORES + lax.axis_index("c")
    base = wid * ROWS_PER_WORKER

    pltpu.sync_copy(x_hbm.at[pl.ds(base, ROWS_PER_WORKER)], idx_v)
    pltpu.sync_copy(pos_hbm, pos_v)

    def fire_g(s, b):
        off = s * MAXLEN
        for sb, sl in SUBS:
            pltpu.async_copy(
                tok_hbm.at[idx_v.at[pl.ds(off + sb, sl)]],
                ring[b].at[pl.ds(sb, sl)],
                gsem[b],
            )

    def wait_g(b):
        pltpu.make_async_copy(tok_hbm.at[pl.ds(0, MAXLEN)], ring[b], gsem[b]).wait()

    def fire_o(s, b):
        pltpu.async_copy(ring[b], out_hbm.at[pl.ds(base + s * MAXLEN, MAXLEN)], osem[b])

    def wait_o(b):
        pltpu.make_async_copy(ring[b], out_hbm.at[pl.ds(base, MAXLEN)], osem[b]).wait()

    def add_pos(b):
        buf = ring[b]

        def rbody(r8, c):
            for u in range(8):
                r = r8 * 8 + u
                for cb in range(DBLK):
                    sl = pl.ds(cb * LANES, LANES)
                    buf[r, sl] = buf[r, sl] + pos_v[r, sl]
            return c

        lax.fori_loop(0, MAXLEN // 8, rbody, 0)

    def step(s, b, prefetch=True, drain=True):
        if prefetch:
            if drain:
                wait_o((b + 2) % RING)
            fire_g(s + 2, (b + 2) % RING)
        wait_g(b)
        add_pos(b)
        fire_o(s, b)

    fire_g(0, 0)
    fire_g(1, 1)
    step(0, 0, drain=False)
    step(1, 1, drain=False)
    step(2, 2)
    step(3, 3)

    def macro_body(m, carry):
        s0 = m * RING
        for b in range(RING):
            step(s0 + b, b)
        return carry

    lax.fori_loop(1, SEQS_PER_WORKER // RING - 1, macro_body, 0)

    s0 = SEQS_PER_WORKER - RING
    step(s0 + 0, 0, prefetch=True)
    step(s0 + 1, 1, prefetch=True)
    step(s0 + 2, 2, prefetch=False)
    step(s0 + 3, 3, prefetch=False)
    for b in range(RING):
        wait_o(b)


def kernel(x, token_emb, pos_emb):
    x_flat = x.reshape(-1).astype(jnp.int32)
    out = _emb_kernel(x_flat, token_emb, pos_emb)
    return out.reshape(BATCH, MAXLEN, EMBED_DIM)
